# Initial kernel scaffold; baseline (speedup 1.0000x reference)
import jax
import jax.numpy as jnp
from jax.experimental import pallas as pl


def kernel(vfeat, efeat, v_reg_weight, v_reg_sum, e_reg_weight, e_reg_sum, in_src, in_dst, W1, b1, Wve, bve, Wev, bev, first_layer, last_layer):
    raise NotImplementedError("write your pallas kernel here")


# speedup vs baseline: 25.8493x; 25.8493x over previous
import jax
import jax.numpy as jnp
from jax import lax
from jax.experimental import pallas as pl
from jax.experimental.pallas import tpu as pltpu
from jax.experimental.pallas import tpu_sc as plsc

D = 128
NC = 2
NS = 16
NW = NC * NS


def _mlp_u_body(vfeat_ref, w1_ref, b1_ref, wve_ref, bve_ref, vrw_ref, u_ref):
    t = jnp.dot(vfeat_ref[...], w1_ref[...],
                preferred_element_type=jnp.float32) + b1_ref[...]
    u = jnp.dot(t, wve_ref[...],
                preferred_element_type=jnp.float32) + bve_ref[...]
    u_ref[...] = vrw_ref[...] * u


def _edge_body(p_ref, ers_ref, erw_ref, wev_ref, bev_ref, fe_ref, v_ref):
    fe = (p_ref[0] + p_ref[1]) / ers_ref[...]
    fe_ref[...] = fe
    v = jnp.dot(fe, wev_ref[...],
                preferred_element_type=jnp.float32) + bev_ref[...]
    v_ref[...] = erw_ref[...] * v


def _vout_body(p_ref, vrs_ref, out_ref):
    out_ref[...] = (p_ref[0] + p_ref[1]) / vrs_ref[...]


def _sc_pass(table, gidx, sidx, acc_rows, nchunk, chunk):
    zeros = jnp.zeros((chunk, D), jnp.float32)
    zr = acc_rows // NS
    kz = zr // chunk

    mesh = plsc.VectorSubcoreMesh(core_axis_name="c", subcore_axis_name="s")

    def body(table_hbm, gidx_hbm, sidx_hbm, zeros_hbm, out_hbm,
             acc, gidx_v, sidx_v, rows_v, sem):
        c = lax.axis_index("c")
        s = lax.axis_index("s")
        wid = c * NS + s
        pltpu.sync_copy(gidx_hbm.at[wid], gidx_v)
        pltpu.sync_copy(sidx_hbm.at[wid], sidx_v)
        pltpu.sync_copy(zeros_hbm, rows_v)
        for k in range(kz):
            pltpu.sync_copy(rows_v, acc.at[pl.ds(s * zr + k * chunk, chunk)])
        plsc.subcore_barrier()

        def chunk_body(j, carry):
            pltpu.async_copy(table_hbm.at[gidx_v.at[j]], rows_v, sem).wait()
            pltpu.sync_copy(rows_v, acc.at[sidx_v.at[j]], add=True)
            return carry

        lax.fori_loop(0, nchunk, chunk_body, 0)
        plsc.subcore_barrier()
        pltpu.sync_copy(acc.at[pl.ds(s * zr, zr)],
                        out_hbm.at[c, pl.ds(s * zr, zr)])

    return pl.kernel(
        body,
        out_type=jax.ShapeDtypeStruct((NC, acc_rows, D), jnp.float32),
        mesh=mesh,
        scratch_types=[
            pltpu.VMEM_SHARED((acc_rows, D), jnp.float32),
            pltpu.VMEM((nchunk, chunk), jnp.int32),
            pltpu.VMEM((nchunk, chunk), jnp.int32),
            pltpu.VMEM((chunk, D), jnp.float32),
            pltpu.SemaphoreType.DMA,
        ],
    )(table, gidx, sidx, zeros)


def kernel(vfeat, efeat, v_reg_weight, v_reg_sum, e_reg_weight, e_reg_sum,
           in_src, in_dst, W1, b1, Wve, bve, Wev, bev,
           first_layer=1, last_layer=1):
    N, D_IN = vfeat.shape
    M = e_reg_sum.shape[0]
    E = in_src.shape[0]

    chunk = 80
    nchunk = E // (NW * chunk)
    gidx = in_src.astype(jnp.int32).reshape(NW, nchunk, chunk)
    sidx = in_dst.astype(jnp.int32).reshape(NW, nchunk, chunk)

    BA = 2000
    u = pl.pallas_call(
        _mlp_u_body,
        grid=(N // BA,),
        in_specs=[
            pl.BlockSpec((BA, D_IN), lambda i: (i, 0)),
            pl.BlockSpec((D_IN, D), lambda i: (0, 0)),
            pl.BlockSpec((1, D), lambda i: (0, 0)),
            pl.BlockSpec((D, D), lambda i: (0, 0)),
            pl.BlockSpec((1, D), lambda i: (0, 0)),
            pl.BlockSpec((BA, 1), lambda i: (i, 0)),
        ],
        out_specs=pl.BlockSpec((BA, D), lambda i: (i, 0)),
        out_shape=jax.ShapeDtypeStruct((N, D), jnp.float32),
    )(vfeat, W1, b1.reshape(1, D), Wve, bve.reshape(1, D), v_reg_weight)

    MP = 5120
    pe = _sc_pass(u, gidx, sidx, MP, nchunk, chunk)

    BB = 1000
    feat_e, v = pl.pallas_call(
        _edge_body,
        grid=(M // BB,),
        in_specs=[
            pl.BlockSpec((NC, BB, D), lambda i: (0, i, 0)),
            pl.BlockSpec((BB, 1), lambda i: (i, 0)),
            pl.BlockSpec((BB, 1), lambda i: (i, 0)),
            pl.BlockSpec((D, D), lambda i: (0, 0)),
            pl.BlockSpec((1, D), lambda i: (0, 0)),
        ],
        out_specs=[
            pl.BlockSpec((BB, D), lambda i: (i, 0)),
            pl.BlockSpec((BB, D), lambda i: (i, 0)),
        ],
        out_shape=[
            jax.ShapeDtypeStruct((M, D), jnp.float32),
            jax.ShapeDtypeStruct((M, D), jnp.float32),
        ],
    )(pe, e_reg_sum, e_reg_weight, Wev, bev.reshape(1, D))

    NP = 10240
    pv = _sc_pass(v, sidx, gidx, NP, nchunk, chunk)

    BC = 2000
    feat_v_out = pl.pallas_call(
        _vout_body,
        grid=(N // BC,),
        in_specs=[
            pl.BlockSpec((NC, BC, D), lambda i: (0, i, 0)),
            pl.BlockSpec((BC, 1), lambda i: (i, 0)),
        ],
        out_specs=pl.BlockSpec((BC, D), lambda i: (i, 0)),
        out_shape=jax.ShapeDtypeStruct((N, D), jnp.float32),
    )(pv, v_reg_sum)

    return (feat_v_out, feat_e)
